# baseline (speedup 1.0000x reference)
import functools

import jax
import jax.numpy as jnp
from jax import lax
from jax.experimental import pallas as pl
from jax.experimental.pallas import tpu as pltpu
from jax.experimental.pallas import tpu_sc as plsc

B = 16
L = 4096
D = 300
DPAD = 304
NTILES = 32
DBLK = 25
DSC = 300
NQ = 4 * (D - DSC)
CL = L // 2
NVR = CL // 16
RING = 4


def _tc_native_body(x_ref, li_ref, lf_ref, o_ref):
    i = pl.program_id(0)
    x = x_ref[...]
    iota_l = lax.broadcasted_iota(jnp.int32, (B, L), 1)
    mask = iota_l < li_ref[...]
    s = jnp.sum(jnp.where(mask[None], x, 0.0), axis=2)
    o_ref[pl.ds(i * DBLK, DBLK), :] = s / lf_ref[...]


def _tc_native(x_t, li, lf):
    return pl.pallas_call(
        _tc_native_body,
        grid=(DSC // DBLK,),
        in_specs=[
            pl.BlockSpec((DBLK, B, L), lambda i: (i, 0, 0)),
            pl.BlockSpec((B, 1), lambda i: (0, 0)),
            pl.BlockSpec((1, B), lambda i: (0, 0)),
        ],
        out_specs=pl.BlockSpec((DSC, B), lambda i: (0, 0)),
        out_shape=jax.ShapeDtypeStruct((DSC, B), jnp.float32),
        compiler_params=pltpu.CompilerParams(
            dimension_semantics=("arbitrary",),
        ),
    )(x_t, li, lf)


@functools.cache
def _make_sc_dslab():
    mesh = plsc.VectorSubcoreMesh(core_axis_name="c", subcore_axis_name="s")
    return functools.partial(
        pl.kernel,
        out_type=jax.ShapeDtypeStruct((NTILES * DPAD * 16,), jnp.float32),
        mesh=mesh,
        compiler_params=pltpu.CompilerParams(needs_layout_passes=False),
        scratch_types=[
            pltpu.VMEM((16,), jnp.int32),
            pltpu.VMEM((RING, 8, CL), jnp.float32),
            pltpu.VMEM((DPAD * 16,), jnp.float32),
            pltpu.SemaphoreType.DMA,
            pltpu.SemaphoreType.DMA,
            pltpu.SemaphoreType.DMA,
            pltpu.SemaphoreType.DMA,
        ],
    )(_sc_dslab_body)


def _sc_dslab_body(x_ref, len_ref, out_ref, len_v, buf, acc,
                   sem0, sem1, sem2, sem3):
    wid = lax.axis_index("s") * 2 + lax.axis_index("c")

    zero = jnp.zeros((16,), jnp.float32)

    def _zero_acc(i, carry):
        acc[pl.ds(i * 16, 16)] = zero
        return carry

    lax.fori_loop(0, DPAD, _zero_acc, 0)

    pltpu.sync_copy(len_ref, len_v)
    lv = len_v[...]
    idx16 = lax.broadcasted_iota(jnp.int32, (16,), 0)
    len_s = [jnp.sum(jnp.where(idx16 == b, lv, 0)) for b in range(B)]

    sems = (sem0, sem1, sem2, sem3)

    def chunk_q(k):
        return 4 * DSC + wid + k * NTILES

    def copy_op(k, p):
        q = chunk_q(k)
        d = q // 4
        tr = lax.rem(q // 2, 2)
        h = lax.rem(q, 2)
        return pltpu.make_async_copy(
            x_ref.at[d, pl.ds(tr * 8, 8), pl.ds(h * CL, CL)],
            buf.at[p],
            sems[p],
        )

    nmine = lax.div(NQ - wid + NTILES - 1, NTILES)

    for pp in range(RING):
        @pl.when(pp < nmine)
        def _prime():
            copy_op(pp, pp).start()

    def chunk_body(k, carry):
        p = lax.rem(k, RING)

        for pp in range(RING):
            @pl.when(p == pp)
            def _waitp():
                copy_op(k, pp).wait()

        q = chunk_q(k)
        d = q // 4
        tr = lax.rem(q // 2, 2)
        h = lax.rem(q, 2)
        lo = h * CL
        acc_v = zero
        for r in range(8):
            len_b = lax.select(tr > 0, len_s[8 + r], len_s[r])
            live = jnp.clip(len_b - lo, 0, CL)
            off_j = jnp.minimum(live // 16, NVR - 1)
            ng = off_j // 8

            def g8(i, accs):
                a0, a1, a2, a3 = accs
                base = i * 8
                v = [buf[p, r, pl.ds((base + t) * 16, 16)] for t in range(8)]
                return (a0 + v[0] + v[4], a1 + v[1] + v[5],
                        a2 + v[2] + v[6], a3 + v[3] + v[7])

            a0, a1, a2, a3 = lax.fori_loop(0, ng, g8, (zero, zero, zero, zero))

            def g1(j, a):
                return a + buf[p, r, pl.ds(j * 16, 16)]

            at = lax.fori_loop(ng * 8, off_j, g1, zero)
            vb = buf[p, r, pl.ds(off_j * 16, 16)]
            at = at + jnp.where(idx16 < (live - off_j * 16), vb, 0.0)
            s_b = jnp.sum((a0 + a1) + (a2 + a3) + at)
            bb = tr * 8 + r
            acc_v = jnp.where(idx16 == bb, acc_v + s_b, acc_v)
        sl = pl.ds(d * 16, 16)
        acc[sl] = acc[sl] + acc_v

        for pp in range(RING):
            @pl.when(jnp.logical_and(k + RING < nmine, p == pp))
            def _refill():
                copy_op(k + RING, pp).start()
        return carry

    lax.fori_loop(0, nmine, chunk_body, 0)

    pltpu.sync_copy(acc, out_ref.at[pl.ds(wid * DPAD * 16, DPAD * 16)])


def _sc_finish_body(p_ref, lf_ref, o_ref):
    s = jnp.sum(p_ref[...], axis=0)
    o_ref[...] = s[DSC:D] / lf_ref[...]


def _sc_finish(partials, lf):
    return pl.pallas_call(
        _sc_finish_body,
        out_shape=jax.ShapeDtypeStruct((D - DSC, B), jnp.float32),
    )(partials.reshape(NTILES, DPAD, 16), lf)


def kernel(sentences, sentence_lengths):
    x_t = jnp.transpose(sentences, (2, 0, 1))
    li = sentence_lengths.reshape(B, 1)
    lf = sentence_lengths.astype(jnp.float32).reshape(1, B)
    return _tc_native(x_t, li, lf).T


# speedup vs baseline: 1.0872x; 1.0872x over previous
import functools

import jax
import jax.numpy as jnp
from jax import lax
from jax.experimental import pallas as pl
from jax.experimental.pallas import tpu as pltpu
from jax.experimental.pallas import tpu_sc as plsc

B = 16
L = 4096
D = 300
DPAD = 304
NTILES = 32
DBLK = 25
DSC = 300
NQ = 4 * (D - DSC)
CL = L // 2
NVR = CL // 16
RING = 4


def _tc_native_body(x_ref, li_ref, lf_ref, o_ref):
    i = pl.program_id(0)
    x = x_ref[...]
    iota_l = lax.broadcasted_iota(jnp.int32, (B, L), 1)
    mask = iota_l < li_ref[...].reshape(B, 1)
    s = jnp.sum(jnp.where(mask[None], x, 0.0), axis=2)
    o_ref[pl.ds(i * DBLK, DBLK), :] = s / lf_ref[...]


def _tc_native(x_t, li, lf):
    return pl.pallas_call(
        _tc_native_body,
        grid=(DSC // DBLK,),
        in_specs=[
            pl.BlockSpec((DBLK, B, L), lambda i: (i, 0, 0)),
            pl.BlockSpec((1, B), lambda i: (0, 0)),
            pl.BlockSpec((1, B), lambda i: (0, 0)),
        ],
        out_specs=pl.BlockSpec((DSC, B), lambda i: (0, 0)),
        out_shape=jax.ShapeDtypeStruct((DSC, B), jnp.float32),
        compiler_params=pltpu.CompilerParams(
            dimension_semantics=("arbitrary",),
        ),
    )(x_t, li, lf)


@functools.cache
def _make_sc_dslab():
    mesh = plsc.VectorSubcoreMesh(core_axis_name="c", subcore_axis_name="s")
    return functools.partial(
        pl.kernel,
        out_type=jax.ShapeDtypeStruct((NTILES * DPAD * 16,), jnp.float32),
        mesh=mesh,
        compiler_params=pltpu.CompilerParams(needs_layout_passes=False),
        scratch_types=[
            pltpu.VMEM((16,), jnp.int32),
            pltpu.VMEM((RING, 8, CL), jnp.float32),
            pltpu.VMEM((DPAD * 16,), jnp.float32),
            pltpu.SemaphoreType.DMA,
            pltpu.SemaphoreType.DMA,
            pltpu.SemaphoreType.DMA,
            pltpu.SemaphoreType.DMA,
        ],
    )(_sc_dslab_body)


def _sc_dslab_body(x_ref, len_ref, out_ref, len_v, buf, acc,
                   sem0, sem1, sem2, sem3):
    wid = lax.axis_index("s") * 2 + lax.axis_index("c")

    zero = jnp.zeros((16,), jnp.float32)

    def _zero_acc(i, carry):
        acc[pl.ds(i * 16, 16)] = zero
        return carry

    lax.fori_loop(0, DPAD, _zero_acc, 0)

    pltpu.sync_copy(len_ref, len_v)
    lv = len_v[...]
    idx16 = lax.broadcasted_iota(jnp.int32, (16,), 0)
    len_s = [jnp.sum(jnp.where(idx16 == b, lv, 0)) for b in range(B)]

    sems = (sem0, sem1, sem2, sem3)

    def chunk_q(k):
        return 4 * DSC + wid + k * NTILES

    def copy_op(k, p):
        q = chunk_q(k)
        d = q // 4
        tr = lax.rem(q // 2, 2)
        h = lax.rem(q, 2)
        return pltpu.make_async_copy(
            x_ref.at[d, pl.ds(tr * 8, 8), pl.ds(h * CL, CL)],
            buf.at[p],
            sems[p],
        )

    nmine = lax.div(NQ - wid + NTILES - 1, NTILES)

    for pp in range(RING):
        @pl.when(pp < nmine)
        def _prime():
            copy_op(pp, pp).start()

    def chunk_body(k, carry):
        p = lax.rem(k, RING)

        for pp in range(RING):
            @pl.when(p == pp)
            def _waitp():
                copy_op(k, pp).wait()

        q = chunk_q(k)
        d = q // 4
        tr = lax.rem(q // 2, 2)
        h = lax.rem(q, 2)
        lo = h * CL
        acc_v = zero
        for r in range(8):
            len_b = lax.select(tr > 0, len_s[8 + r], len_s[r])
            live = jnp.clip(len_b - lo, 0, CL)
            off_j = jnp.minimum(live // 16, NVR - 1)
            ng = off_j // 8

            def g8(i, accs):
                a0, a1, a2, a3 = accs
                base = i * 8
                v = [buf[p, r, pl.ds((base + t) * 16, 16)] for t in range(8)]
                return (a0 + v[0] + v[4], a1 + v[1] + v[5],
                        a2 + v[2] + v[6], a3 + v[3] + v[7])

            a0, a1, a2, a3 = lax.fori_loop(0, ng, g8, (zero, zero, zero, zero))

            def g1(j, a):
                return a + buf[p, r, pl.ds(j * 16, 16)]

            at = lax.fori_loop(ng * 8, off_j, g1, zero)
            vb = buf[p, r, pl.ds(off_j * 16, 16)]
            at = at + jnp.where(idx16 < (live - off_j * 16), vb, 0.0)
            s_b = jnp.sum((a0 + a1) + (a2 + a3) + at)
            bb = tr * 8 + r
            acc_v = jnp.where(idx16 == bb, acc_v + s_b, acc_v)
        sl = pl.ds(d * 16, 16)
        acc[sl] = acc[sl] + acc_v

        for pp in range(RING):
            @pl.when(jnp.logical_and(k + RING < nmine, p == pp))
            def _refill():
                copy_op(k + RING, pp).start()
        return carry

    lax.fori_loop(0, nmine, chunk_body, 0)

    pltpu.sync_copy(acc, out_ref.at[pl.ds(wid * DPAD * 16, DPAD * 16)])


def _sc_finish_body(p_ref, lf_ref, o_ref):
    s = jnp.sum(p_ref[...], axis=0)
    o_ref[...] = s[DSC:D] / lf_ref[...]


def _sc_finish(partials, lf):
    return pl.pallas_call(
        _sc_finish_body,
        out_shape=jax.ShapeDtypeStruct((D - DSC, B), jnp.float32),
    )(partials.reshape(NTILES, DPAD, 16), lf)


def kernel(sentences, sentence_lengths):
    x_t = jnp.transpose(sentences, (2, 0, 1))
    li = sentence_lengths.reshape(1, B)
    lf = sentence_lengths.astype(jnp.float32).reshape(1, B)
    return _tc_native(x_t, li, lf).T
